# Initial kernel scaffold; baseline (speedup 1.0000x reference)
import jax
import jax.numpy as jnp
from jax.experimental import pallas as pl


def kernel(x, edge_index, W1, b1, W2, b2, Wl1, bl1, Wl2, bl2):
    raise NotImplementedError("write your pallas kernel here")


# speedup vs baseline: 16.4128x; 16.4128x over previous
import functools

import jax
import jax.numpy as jnp
from jax import lax
from jax.experimental import pallas as pl
from jax.experimental.pallas import tpu as pltpu
from jax.experimental.pallas import tpu_sc as plsc

N = 50000
NP = 50176
E = 1600000
D = 32

NCORES = 2
NTILES = 16
ROWS_T = NP // NTILES
E_CORE = E // NCORES
E_TILE = E_CORE // NTILES
C = 80
NCHUNK = E_TILE // C
RQ = ROWS_T // 4

_mesh = plsc.VectorSubcoreMesh(core_axis_name="c", subcore_axis_name="s")


@functools.partial(
    pl.kernel,
    out_type=jax.ShapeDtypeStruct((NCORES * NP,), jnp.float32),
    mesh=_mesh,
    compiler_params=pltpu.CompilerParams(use_tc_tiling_on_sc=False),
    scratch_types=[
        pltpu.VMEM((C,), jnp.int32),
        pltpu.VMEM((C,), jnp.float32),
        pltpu.VMEM((ROWS_T,), jnp.float32),
        pltpu.VMEM_SHARED((NP,), jnp.float32),
    ],
)
def _sc_degree(dst_hbm, zeros_hbm, out_hbm, idx_v, ones_v, stg, deg_sh):
    c = lax.axis_index("c")
    s = lax.axis_index("s")
    pltpu.sync_copy(zeros_hbm.at[pl.ds(s * ROWS_T, ROWS_T)], stg)
    pltpu.sync_copy(stg, deg_sh.at[pl.ds(s * ROWS_T, ROWS_T)])
    for i in range(C // 16):
        ones_v[pl.ds(i * 16, 16)] = jnp.full((16,), 1.0, jnp.float32)
    plsc.subcore_barrier()
    base0 = c * E_CORE + s * E_TILE

    def chunk(j, carry):
        b = base0 + j * C
        pltpu.sync_copy(dst_hbm.at[pl.ds(b, C)], idx_v)
        pltpu.sync_copy(ones_v, deg_sh.at[idx_v], add=True)
        return carry

    lax.fori_loop(0, NCHUNK, chunk, 0)
    plsc.subcore_barrier()
    pltpu.sync_copy(deg_sh.at[pl.ds(s * ROWS_T, ROWS_T)], stg)
    pltpu.sync_copy(stg, out_hbm.at[pl.ds(c * NP + s * ROWS_T, ROWS_T)])


@functools.partial(
    pl.kernel,
    out_type=jax.ShapeDtypeStruct((NCORES, NP, D), jnp.float32),
    mesh=_mesh,
    compiler_params=pltpu.CompilerParams(use_tc_tiling_on_sc=False),
    scratch_types=[
        pltpu.VMEM((C,), jnp.int32),
        pltpu.VMEM((C,), jnp.int32),
        pltpu.VMEM((C, D), jnp.float32),
        pltpu.VMEM((RQ, D), jnp.float32),
        pltpu.VMEM_SHARED((NP, D), jnp.float32),
        pltpu.SemaphoreType.DMA,
    ],
)
def _sc_segsum(tbl_hbm, src_hbm, dst_hbm, zeros_hbm, out_hbm,
               sidx, didx, rows_v, stg, acc_sh, sem):
    c = lax.axis_index("c")
    s = lax.axis_index("s")
    for q in range(ROWS_T // RQ):
        r0 = s * ROWS_T + q * RQ
        pltpu.sync_copy(zeros_hbm.at[pl.ds(r0, RQ)], stg)
        pltpu.sync_copy(stg, acc_sh.at[pl.ds(r0, RQ)])
    plsc.subcore_barrier()
    base0 = c * E_CORE + s * E_TILE

    def chunk(j, carry):
        b = base0 + j * C
        pltpu.sync_copy(src_hbm.at[pl.ds(b, C)], sidx)
        pltpu.sync_copy(dst_hbm.at[pl.ds(b, C)], didx)
        pltpu.async_copy(tbl_hbm.at[sidx], rows_v, sem).wait()
        pltpu.sync_copy(rows_v, acc_sh.at[didx], add=True)
        return carry

    lax.fori_loop(0, NCHUNK, chunk, 0)
    plsc.subcore_barrier()
    for q in range(ROWS_T // RQ):
        r0 = s * ROWS_T + q * RQ
        pltpu.sync_copy(acc_sh.at[pl.ds(r0, RQ)], stg)
        pltpu.sync_copy(stg, out_hbm.at[c, pl.ds(r0, RQ)])


R = 512
G = NP // R


def _tc1_body(x_ref, degp_ref, W1_ref, b1_ref, hp_ref, aux_ref):
    deg = degp_ref[0, :] + degp_ref[1, :] + 1.0
    dis = lax.rsqrt(deg)[:, None]
    h = jnp.dot(x_ref[...].astype(jnp.bfloat16), W1_ref[...].astype(jnp.bfloat16),
                preferred_element_type=jnp.float32)
    hp_ref[...] = h * dis
    aux_ref[...] = h * (dis * dis) + b1_ref[...]


_tc1 = pl.pallas_call(
    _tc1_body,
    grid=(G,),
    in_specs=[
        pl.BlockSpec((R, 22), lambda i: (i, 0)),
        pl.BlockSpec((NCORES, R), lambda i: (0, i)),
        pl.BlockSpec((22, D), lambda i: (0, 0)),
        pl.BlockSpec((1, D), lambda i: (0, 0)),
    ],
    out_specs=[
        pl.BlockSpec((R, D), lambda i: (i, 0)),
        pl.BlockSpec((R, D), lambda i: (i, 0)),
    ],
    out_shape=[
        jax.ShapeDtypeStruct((NP, D), jnp.float32),
        jax.ShapeDtypeStruct((NP, D), jnp.float32),
    ],
)


def _tc2_body(accp_ref, aux1_ref, degp_ref, W2_ref, b2_ref, hp2_ref, aux2_ref):
    deg = degp_ref[0, :] + degp_ref[1, :] + 1.0
    dis = lax.rsqrt(deg)[:, None]
    acc = accp_ref[0] + accp_ref[1]
    out1 = jnp.maximum(dis * acc + aux1_ref[...], 0.0)
    h2 = jnp.dot(out1.astype(jnp.bfloat16), W2_ref[...].astype(jnp.bfloat16),
                 preferred_element_type=jnp.float32)
    hp2_ref[...] = h2 * dis
    aux2_ref[...] = h2 * (dis * dis) + b2_ref[...]


_tc2 = pl.pallas_call(
    _tc2_body,
    grid=(G,),
    in_specs=[
        pl.BlockSpec((NCORES, R, D), lambda i: (0, i, 0)),
        pl.BlockSpec((R, D), lambda i: (i, 0)),
        pl.BlockSpec((NCORES, R), lambda i: (0, i)),
        pl.BlockSpec((D, D), lambda i: (0, 0)),
        pl.BlockSpec((1, D), lambda i: (0, 0)),
    ],
    out_specs=[
        pl.BlockSpec((R, D), lambda i: (i, 0)),
        pl.BlockSpec((R, D), lambda i: (i, 0)),
    ],
    out_shape=[
        jax.ShapeDtypeStruct((NP, D), jnp.float32),
        jax.ShapeDtypeStruct((NP, D), jnp.float32),
    ],
)


def _tc3_body(accp_ref, aux2_ref, degp_ref, Wl1_ref, bl1_ref, Wl2_ref,
              bl2_ref, y_ref):
    deg = degp_ref[0, :] + degp_ref[1, :] + 1.0
    dis = lax.rsqrt(deg)[:, None]
    out2 = jnp.maximum(dis * (accp_ref[0] + accp_ref[1]) + aux2_ref[...], 0.0)
    m = jnp.maximum(
        jnp.dot(out2.astype(jnp.bfloat16), Wl1_ref[...].astype(jnp.bfloat16),
                preferred_element_type=jnp.float32)
        + bl1_ref[...], 0.0)
    y_ref[...] = (jnp.dot(m.astype(jnp.bfloat16), Wl2_ref[...].astype(jnp.bfloat16),
                          preferred_element_type=jnp.float32)
                  + bl2_ref[...])


_tc3 = pl.pallas_call(
    _tc3_body,
    grid=(G,),
    in_specs=[
        pl.BlockSpec((NCORES, R, D), lambda i: (0, i, 0)),
        pl.BlockSpec((R, D), lambda i: (i, 0)),
        pl.BlockSpec((NCORES, R), lambda i: (0, i)),
        pl.BlockSpec((D, 10), lambda i: (0, 0)),
        pl.BlockSpec((1, 10), lambda i: (0, 0)),
        pl.BlockSpec((10, 2), lambda i: (0, 0)),
        pl.BlockSpec((1, 2), lambda i: (0, 0)),
    ],
    out_specs=pl.BlockSpec((R, 2), lambda i: (i, 0)),
    out_shape=jax.ShapeDtypeStruct((NP, 2), jnp.float32),
)


def kernel(x, edge_index, W1, b1, W2, b2, Wl1, bl1, Wl2, bl2):
    src = edge_index[0].astype(jnp.int32)
    dst = edge_index[1].astype(jnp.int32)
    x_p = jnp.pad(x, ((0, NP - N), (0, 0)))
    zeros_d = jnp.zeros((NP, D), jnp.float32)
    zeros_1 = jnp.zeros((NP,), jnp.float32)

    degp = _sc_degree(dst, zeros_1).reshape(NCORES, NP)
    hp1, aux1 = _tc1(x_p, degp, W1, b1.reshape(1, D))
    acc1 = _sc_segsum(hp1, src, dst, zeros_d)
    W2p = jnp.pad(W2, ((0, 0), (0, D - 20)))
    b2p = jnp.pad(b2, (0, D - 20)).reshape(1, D)
    hp2, aux2 = _tc2(acc1, aux1, degp, W2p, b2p)
    acc2 = _sc_segsum(hp2, src, dst, zeros_d)
    Wl1p = jnp.pad(Wl1, ((0, D - 20), (0, 0)))
    y = _tc3(acc2, aux2, degp, Wl1p, bl1.reshape(1, 10), Wl2,
             bl2.reshape(1, 2))
    return y[:N]


# speedup vs baseline: 53.6738x; 3.2702x over previous
import functools

import jax
import jax.numpy as jnp
from jax import lax
from jax.experimental import pallas as pl
from jax.experimental.pallas import tpu as pltpu
from jax.experimental.pallas import tpu_sc as plsc

N = 50000
NP = 50176
E = 1600000
D = 32

NCORES = 2
NTILES = 16
NW = NCORES * NTILES
ROWS_T = NP // NTILES
RQ = ROWS_T // 16

CC = 128
NB = 2
EG = NB * CC
NG = 196
EP_TILE = NG * EG
ROWS_E = EP_TILE // CC
E_PAD = NW * EP_TILE
PAD_BASE = 50048

_mesh = plsc.VectorSubcoreMesh(core_axis_name="c", subcore_axis_name="s")


def _gather(tbl_hbm, idxrow, rows, sem):
    return pltpu.make_async_copy(tbl_hbm.at[idxrow], rows, sem)


@functools.partial(
    pl.kernel,
    out_type=jax.ShapeDtypeStruct((NCORES * NP,), jnp.float32),
    mesh=_mesh,
    compiler_params=pltpu.CompilerParams(use_tc_tiling_on_sc=False),
    scratch_types=[
        pltpu.VMEM((NB, CC), jnp.int32),
        pltpu.VMEM((NB, CC), jnp.int32),
        pltpu.VMEM((CC,), jnp.float32),
        pltpu.VMEM((ROWS_T,), jnp.float32),
        pltpu.VMEM_SHARED((NP,), jnp.float32),
        pltpu.SemaphoreType.DMA,
        pltpu.SemaphoreType.DMA,
    ],
)
def _sc_degree(dst2d_hbm, zeros_hbm, out_hbm, idx0, idx1, ones_v, stg, deg_sh,
               semi0, semi1):
    c = lax.axis_index("c")
    s = lax.axis_index("s")
    w = c * NTILES + s
    pltpu.sync_copy(zeros_hbm, stg)
    pltpu.sync_copy(stg, deg_sh.at[pl.ds(s * ROWS_T, ROWS_T)])
    for i in range(CC // 16):
        ones_v[pl.ds(i * 16, 16)] = jnp.full((16,), 1.0, jnp.float32)
    plsc.subcore_barrier()

    r0 = w * ROWS_E
    pltpu.sync_copy(dst2d_hbm.at[pl.ds(r0, NB)], idx0)
    pltpu.async_copy(dst2d_hbm.at[pl.ds(r0 + NB, NB)], idx1, semi1)

    def phase(g, idx, oidx, semo, semself):
        @pl.when(g + 1 < NG)
        def _():
            pltpu.make_async_copy(
                dst2d_hbm.at[pl.ds(r0 + (g + 1) * NB, NB)], oidx, semo).wait()

        for b in range(NB):
            pltpu.sync_copy(ones_v, deg_sh.at[idx.at[b]], add=True)

        @pl.when(g + 2 < NG)
        def _():
            pltpu.async_copy(
                dst2d_hbm.at[pl.ds(r0 + (g + 2) * NB, NB)], idx, semself)

    def pair(k, carry):
        g = 2 * k
        phase(g, idx0, idx1, semi1, semi0)
        phase(g + 1, idx1, idx0, semi0, semi1)
        return carry

    lax.fori_loop(0, NG // 2, pair, 0)
    plsc.subcore_barrier()
    pltpu.sync_copy(deg_sh.at[pl.ds(s * ROWS_T, ROWS_T)], stg)
    pltpu.sync_copy(stg, out_hbm.at[pl.ds(c * NP + s * ROWS_T, ROWS_T)])


@functools.partial(
    pl.kernel,
    out_type=jax.ShapeDtypeStruct((NCORES, NP, D), jnp.float32),
    mesh=_mesh,
    compiler_params=pltpu.CompilerParams(use_tc_tiling_on_sc=False),
    scratch_types=[
        pltpu.VMEM((NB, CC), jnp.int32),
        pltpu.VMEM((NB, CC), jnp.int32),
        pltpu.VMEM((NB, CC), jnp.int32),
        pltpu.VMEM((NB, CC), jnp.int32),
        pltpu.VMEM((NB, CC, D), jnp.float32),
        pltpu.VMEM((NB, CC, D), jnp.float32),
        pltpu.VMEM((RQ, D), jnp.float32),
        pltpu.VMEM_SHARED((NP, D), jnp.float32),
        pltpu.SemaphoreType.DMA,
        pltpu.SemaphoreType.DMA,
        pltpu.SemaphoreType.DMA,
        pltpu.SemaphoreType.DMA,
    ],
)
def _sc_segsum(tbl_hbm, src2d_hbm, dst2d_hbm, zeros_hbm, out_hbm,
               sidx0, sidx1, didx0, didx1, rows0, rows1, stg, acc_sh,
               semi0, semi1, semg0, semg1):
    c = lax.axis_index("c")
    s = lax.axis_index("s")
    w = c * NTILES + s
    pltpu.sync_copy(zeros_hbm, stg)
    for q in range(ROWS_T // RQ):
        pltpu.sync_copy(stg, acc_sh.at[pl.ds(s * ROWS_T + q * RQ, RQ)])
    plsc.subcore_barrier()

    r0 = w * ROWS_E

    def ldidx(g, sidx, didx, sem):
        pltpu.async_copy(src2d_hbm.at[pl.ds(r0 + g * NB, NB)], sidx, sem)
        pltpu.async_copy(dst2d_hbm.at[pl.ds(r0 + g * NB, NB)], didx, sem)

    def wtidx(g, sidx, didx, sem):
        pltpu.make_async_copy(
            src2d_hbm.at[pl.ds(r0 + g * NB, NB)], sidx, sem).wait()
        pltpu.make_async_copy(
            dst2d_hbm.at[pl.ds(r0 + g * NB, NB)], didx, sem).wait()

    pltpu.sync_copy(src2d_hbm.at[pl.ds(r0, NB)], sidx0)
    pltpu.sync_copy(dst2d_hbm.at[pl.ds(r0, NB)], didx0)
    for b in range(NB):
        _gather(tbl_hbm, sidx0.at[b], rows0.at[b], semg0).start()
    ldidx(1, sidx1, didx1, semi1)

    def phase(g, sidx, didx, rows, osidx, odidx, orows,
              semio, semgo, semiself, semgself):
        @pl.when(g + 1 < NG)
        def _():
            wtidx(g + 1, osidx, odidx, semio)
            for b in range(NB):
                _gather(tbl_hbm, osidx.at[b], orows.at[b], semgo).start()

        for b in range(NB):
            _gather(tbl_hbm, sidx.at[b], rows.at[b], semgself).wait()
        for b in range(NB):
            pltpu.sync_copy(rows.at[b], acc_sh.at[didx.at[b]], add=True)

        @pl.when(g + 2 < NG)
        def _():
            ldidx(g + 2, sidx, didx, semiself)

    def pair(k, carry):
        g = 2 * k
        phase(g, sidx0, didx0, rows0, sidx1, didx1, rows1,
              semi1, semg1, semi0, semg0)
        phase(g + 1, sidx1, didx1, rows1, sidx0, didx0, rows0,
              semi0, semg0, semi1, semg1)
        return carry

    lax.fori_loop(0, NG // 2, pair, 0)
    plsc.subcore_barrier()
    for q in range(ROWS_T // RQ):
        r = s * ROWS_T + q * RQ
        pltpu.sync_copy(acc_sh.at[pl.ds(r, RQ)], stg)
        pltpu.sync_copy(stg, out_hbm.at[c, pl.ds(r, RQ)])


R = 3584
G = NP // R


def _tc0_body(x_ref, W1_ref, h_ref):
    h_ref[...] = jnp.dot(x_ref[...].astype(jnp.bfloat16),
                         W1_ref[...].astype(jnp.bfloat16),
                         preferred_element_type=jnp.float32)


_tc0 = pl.pallas_call(
    _tc0_body,
    grid=(G,),
    in_specs=[
        pl.BlockSpec((R, 22), lambda i: (i, 0)),
        pl.BlockSpec((22, D), lambda i: (0, 0)),
    ],
    out_specs=pl.BlockSpec((R, D), lambda i: (i, 0)),
    out_shape=jax.ShapeDtypeStruct((NP, D), jnp.float32),
)


def _tc1_body(h_ref, degp_ref, b1_ref, hp_ref, aux_ref):
    deg = degp_ref[0, :] + degp_ref[1, :] + 1.0
    dis = lax.rsqrt(deg)[:, None]
    h = h_ref[...]
    hp_ref[...] = h * dis
    aux_ref[...] = h * (dis * dis) + b1_ref[...]


_tc1 = pl.pallas_call(
    _tc1_body,
    grid=(G,),
    in_specs=[
        pl.BlockSpec((R, D), lambda i: (i, 0)),
        pl.BlockSpec((NCORES, R), lambda i: (0, i)),
        pl.BlockSpec((1, D), lambda i: (0, 0)),
    ],
    out_specs=[
        pl.BlockSpec((R, D), lambda i: (i, 0)),
        pl.BlockSpec((R, D), lambda i: (i, 0)),
    ],
    out_shape=[
        jax.ShapeDtypeStruct((NP, D), jnp.float32),
        jax.ShapeDtypeStruct((NP, D), jnp.float32),
    ],
)


def _tc2_body(accp_ref, aux1_ref, degp_ref, W2_ref, b2_ref, hp2_ref, aux2_ref):
    deg = degp_ref[0, :] + degp_ref[1, :] + 1.0
    dis = lax.rsqrt(deg)[:, None]
    acc = accp_ref[0] + accp_ref[1]
    out1 = jnp.maximum(dis * acc + aux1_ref[...], 0.0)
    h2 = jnp.dot(out1.astype(jnp.bfloat16), W2_ref[...].astype(jnp.bfloat16),
                 preferred_element_type=jnp.float32)
    hp2_ref[...] = h2 * dis
    aux2_ref[...] = h2 * (dis * dis) + b2_ref[...]


_tc2 = pl.pallas_call(
    _tc2_body,
    grid=(G,),
    in_specs=[
        pl.BlockSpec((NCORES, R, D), lambda i: (0, i, 0)),
        pl.BlockSpec((R, D), lambda i: (i, 0)),
        pl.BlockSpec((NCORES, R), lambda i: (0, i)),
        pl.BlockSpec((D, D), lambda i: (0, 0)),
        pl.BlockSpec((1, D), lambda i: (0, 0)),
    ],
    out_specs=[
        pl.BlockSpec((R, D), lambda i: (i, 0)),
        pl.BlockSpec((R, D), lambda i: (i, 0)),
    ],
    out_shape=[
        jax.ShapeDtypeStruct((NP, D), jnp.float32),
        jax.ShapeDtypeStruct((NP, D), jnp.float32),
    ],
)


def _tc3_body(accp_ref, aux2_ref, degp_ref, Wl1_ref, bl1_ref, Wl2_ref,
              bl2_ref, y_ref):
    deg = degp_ref[0, :] + degp_ref[1, :] + 1.0
    dis = lax.rsqrt(deg)[:, None]
    out2 = jnp.maximum(dis * (accp_ref[0] + accp_ref[1]) + aux2_ref[...], 0.0)
    m = jnp.maximum(
        jnp.dot(out2.astype(jnp.bfloat16), Wl1_ref[...].astype(jnp.bfloat16),
                preferred_element_type=jnp.float32)
        + bl1_ref[...], 0.0)
    y_ref[...] = (jnp.dot(m.astype(jnp.bfloat16),
                          Wl2_ref[...].astype(jnp.bfloat16),
                          preferred_element_type=jnp.float32)
                  + bl2_ref[...])


_tc3 = pl.pallas_call(
    _tc3_body,
    grid=(G,),
    in_specs=[
        pl.BlockSpec((NCORES, R, D), lambda i: (0, i, 0)),
        pl.BlockSpec((R, D), lambda i: (i, 0)),
        pl.BlockSpec((NCORES, R), lambda i: (0, i)),
        pl.BlockSpec((D, 10), lambda i: (0, 0)),
        pl.BlockSpec((1, 10), lambda i: (0, 0)),
        pl.BlockSpec((10, 2), lambda i: (0, 0)),
        pl.BlockSpec((1, 2), lambda i: (0, 0)),
    ],
    out_specs=pl.BlockSpec((R, 2), lambda i: (i, 0)),
    out_shape=jax.ShapeDtypeStruct((N, 2), jnp.float32),
)


def kernel(x, edge_index, W1, b1, W2, b2, Wl1, bl1, Wl2, bl2):
    src = edge_index[0].astype(jnp.int32)
    dst = edge_index[1].astype(jnp.int32)
    pad_idx = PAD_BASE + (jnp.arange(E_PAD - E, dtype=jnp.int32) % 128)
    src2d = jnp.concatenate([src, pad_idx]).reshape(NW * ROWS_E, CC)
    dst2d = jnp.concatenate([dst, pad_idx]).reshape(NW * ROWS_E, CC)
    x_p = jnp.pad(x, ((0, NP - N), (0, 0)))
    zeros_d = jnp.zeros((RQ, D), jnp.float32)
    zeros_1 = jnp.zeros((ROWS_T,), jnp.float32)

    degp = _sc_degree(dst2d, zeros_1).reshape(NCORES, NP)
    h1 = _tc0(x_p, W1)
    hp1, aux1 = _tc1(h1, degp, b1.reshape(1, D))
    acc1 = _sc_segsum(hp1, src2d, dst2d, zeros_d)
    W2p = jnp.pad(W2, ((0, 0), (0, D - 20)))
    b2p = jnp.pad(b2, (0, D - 20)).reshape(1, D)
    hp2, aux2 = _tc2(acc1, aux1, degp, W2p, b2p)
    acc2 = _sc_segsum(hp2, src2d, dst2d, zeros_d)
    Wl1p = jnp.pad(Wl1, ((0, D - 20), (0, 0)))
    return _tc3(acc2, aux2, degp, Wl1p, bl1.reshape(1, 10), Wl2,
                bl2.reshape(1, 2))


# speedup vs baseline: 60.6869x; 1.1307x over previous
import functools

import jax
import jax.numpy as jnp
from jax import lax
from jax.experimental import pallas as pl
from jax.experimental.pallas import tpu as pltpu
from jax.experimental.pallas import tpu_sc as plsc

N = 50000
NP = 50176
E = 1600000
D = 32

NCORES = 2
NTILES = 16
NW = NCORES * NTILES
ROWS_T = NP // NTILES
RQ = ROWS_T // 32

CC = 128
NB = 3
NROWS_E = E // CC
ROWS_W = 390
NG = ROWS_W // NB
NTAIL = NROWS_E - NW * ROWS_W

_mesh = plsc.VectorSubcoreMesh(core_axis_name="c", subcore_axis_name="s")


def _gather(tbl_hbm, idxrow, rows, sem):
    return pltpu.make_async_copy(tbl_hbm.at[idxrow], rows, sem)


@functools.partial(
    pl.kernel,
    out_type=jax.ShapeDtypeStruct((NCORES * NP,), jnp.float32),
    mesh=_mesh,
    compiler_params=pltpu.CompilerParams(use_tc_tiling_on_sc=False),
    scratch_types=[
        pltpu.VMEM((NB, CC), jnp.int32),
        pltpu.VMEM((NB, CC), jnp.int32),
        pltpu.VMEM((CC,), jnp.float32),
        pltpu.VMEM((ROWS_T,), jnp.float32),
        pltpu.VMEM_SHARED((NP,), jnp.float32),
        pltpu.SemaphoreType.DMA,
        pltpu.SemaphoreType.DMA,
    ],
)
def _sc_degree(dst2d_hbm, zeros_hbm, out_hbm, idx0, idx1, ones_v, stg, deg_sh,
               semi0, semi1):
    c = lax.axis_index("c")
    s = lax.axis_index("s")
    w = c * NTILES + s
    pltpu.sync_copy(zeros_hbm, stg)
    pltpu.sync_copy(stg, deg_sh.at[pl.ds(s * ROWS_T, ROWS_T)])
    for i in range(CC // 16):
        ones_v[pl.ds(i * 16, 16)] = jnp.full((16,), 1.0, jnp.float32)
    plsc.subcore_barrier()

    r0 = w * ROWS_W + jnp.minimum(w, NTAIL)
    pltpu.sync_copy(dst2d_hbm.at[pl.ds(r0, NB)], idx0)
    pltpu.async_copy(dst2d_hbm.at[pl.ds(r0 + NB, NB)], idx1, semi1)

    def phase(g, idx, oidx, semo, semself):
        @pl.when(g + 1 < NG)
        def _():
            pltpu.make_async_copy(
                dst2d_hbm.at[pl.ds(r0 + (g + 1) * NB, NB)], oidx, semo).wait()

        for b in range(NB):
            pltpu.sync_copy(ones_v, deg_sh.at[idx.at[b]], add=True)

        @pl.when(g + 2 < NG)
        def _():
            pltpu.async_copy(
                dst2d_hbm.at[pl.ds(r0 + (g + 2) * NB, NB)], idx, semself)

    def pair(k, carry):
        g = 2 * k
        phase(g, idx0, idx1, semi1, semi0)
        phase(g + 1, idx1, idx0, semi0, semi1)
        return carry

    lax.fori_loop(0, NG // 2, pair, 0)

    @pl.when(w < NTAIL)
    def _():
        pltpu.sync_copy(dst2d_hbm.at[pl.ds(r0 + ROWS_W, 1)],
                        idx0.at[pl.ds(0, 1)])
        pltpu.sync_copy(ones_v, deg_sh.at[idx0.at[0]], add=True)

    plsc.subcore_barrier()
    pltpu.sync_copy(deg_sh.at[pl.ds(s * ROWS_T, ROWS_T)], stg)
    pltpu.sync_copy(stg, out_hbm.at[pl.ds(c * NP + s * ROWS_T, ROWS_T)])


@functools.partial(
    pl.kernel,
    out_type=jax.ShapeDtypeStruct((NCORES, NP, D), jnp.float32),
    mesh=_mesh,
    compiler_params=pltpu.CompilerParams(use_tc_tiling_on_sc=False),
    scratch_types=[
        pltpu.VMEM((NB, CC), jnp.int32),
        pltpu.VMEM((NB, CC), jnp.int32),
        pltpu.VMEM((NB, CC), jnp.int32),
        pltpu.VMEM((NB, CC), jnp.int32),
        pltpu.VMEM((NB, CC, D), jnp.float32),
        pltpu.VMEM((NB, CC, D), jnp.float32),
        pltpu.VMEM((RQ, D), jnp.float32),
        pltpu.VMEM_SHARED((NP, D), jnp.float32),
        pltpu.SemaphoreType.DMA,
        pltpu.SemaphoreType.DMA,
        pltpu.SemaphoreType.DMA,
        pltpu.SemaphoreType.DMA,
    ],
)
def _sc_segsum(tbl_hbm, src2d_hbm, dst2d_hbm, zeros_hbm, out_hbm,
               sidx0, sidx1, didx0, didx1, rows0, rows1, stg, acc_sh,
               semi0, semi1, semg0, semg1):
    c = lax.axis_index("c")
    s = lax.axis_index("s")
    w = c * NTILES + s
    pltpu.sync_copy(zeros_hbm, stg)
    for q in range(ROWS_T // RQ):
        pltpu.sync_copy(stg, acc_sh.at[pl.ds(s * ROWS_T + q * RQ, RQ)])
    plsc.subcore_barrier()

    r0 = w * ROWS_W + jnp.minimum(w, NTAIL)

    def ldidx(g, sidx, didx, sem):
        pltpu.async_copy(src2d_hbm.at[pl.ds(r0 + g * NB, NB)], sidx, sem)
        pltpu.async_copy(dst2d_hbm.at[pl.ds(r0 + g * NB, NB)], didx, sem)

    def wtidx(g, sidx, didx, sem):
        pltpu.make_async_copy(
            src2d_hbm.at[pl.ds(r0 + g * NB, NB)], sidx, sem).wait()
        pltpu.make_async_copy(
            dst2d_hbm.at[pl.ds(r0 + g * NB, NB)], didx, sem).wait()

    pltpu.sync_copy(src2d_hbm.at[pl.ds(r0, NB)], sidx0)
    pltpu.sync_copy(dst2d_hbm.at[pl.ds(r0, NB)], didx0)
    for b in range(NB):
        _gather(tbl_hbm, sidx0.at[b], rows0.at[b], semg0).start()
    ldidx(1, sidx1, didx1, semi1)

    def phase(g, sidx, didx, rows, osidx, odidx, orows,
              semio, semgo, semiself, semgself):
        @pl.when(g + 1 < NG)
        def _():
            wtidx(g + 1, osidx, odidx, semio)
            for b in range(NB):
                _gather(tbl_hbm, osidx.at[b], orows.at[b], semgo).start()

        for b in range(NB):
            _gather(tbl_hbm, sidx.at[b], rows.at[b], semgself).wait()
        for b in range(NB):
            pltpu.sync_copy(rows.at[b], acc_sh.at[didx.at[b]], add=True)

        @pl.when(g + 2 < NG)
        def _():
            ldidx(g + 2, sidx, didx, semiself)

    def pair(k, carry):
        g = 2 * k
        phase(g, sidx0, didx0, rows0, sidx1, didx1, rows1,
              semi1, semg1, semi0, semg0)
        phase(g + 1, sidx1, didx1, rows1, sidx0, didx0, rows0,
              semi0, semg0, semi1, semg1)
        return carry

    lax.fori_loop(0, NG // 2, pair, 0)

    @pl.when(w < NTAIL)
    def _():
        pltpu.sync_copy(src2d_hbm.at[pl.ds(r0 + ROWS_W, 1)],
                        sidx0.at[pl.ds(0, 1)])
        pltpu.sync_copy(dst2d_hbm.at[pl.ds(r0 + ROWS_W, 1)],
                        didx0.at[pl.ds(0, 1)])
        _gather(tbl_hbm, sidx0.at[0], rows0.at[0], semg0).start()
        _gather(tbl_hbm, sidx0.at[0], rows0.at[0], semg0).wait()
        pltpu.sync_copy(rows0.at[0], acc_sh.at[didx0.at[0]], add=True)

    plsc.subcore_barrier()
    for q in range(ROWS_T // RQ):
        r = s * ROWS_T + q * RQ
        pltpu.sync_copy(acc_sh.at[pl.ds(r, RQ)], stg)
        pltpu.sync_copy(stg, out_hbm.at[c, pl.ds(r, RQ)])


R = 3584
G = NP // R


def _tc0_body(x_ref, W1_ref, h_ref):
    h_ref[...] = jnp.dot(x_ref[...].astype(jnp.bfloat16),
                         W1_ref[...].astype(jnp.bfloat16),
                         preferred_element_type=jnp.float32)


_tc0 = pl.pallas_call(
    _tc0_body,
    grid=(G,),
    in_specs=[
        pl.BlockSpec((R, 22), lambda i: (i, 0)),
        pl.BlockSpec((22, D), lambda i: (0, 0)),
    ],
    out_specs=pl.BlockSpec((R, D), lambda i: (i, 0)),
    out_shape=jax.ShapeDtypeStruct((NP, D), jnp.float32),
)


def _tc1_body(h_ref, degp_ref, b1_ref, hp_ref, aux_ref):
    deg = degp_ref[0, :] + degp_ref[1, :] + 1.0
    dis = lax.rsqrt(deg)[:, None]
    h = h_ref[...]
    hp_ref[...] = h * dis
    aux_ref[...] = h * (dis * dis) + b1_ref[...]


_tc1 = pl.pallas_call(
    _tc1_body,
    grid=(G,),
    in_specs=[
        pl.BlockSpec((R, D), lambda i: (i, 0)),
        pl.BlockSpec((NCORES, R), lambda i: (0, i)),
        pl.BlockSpec((1, D), lambda i: (0, 0)),
    ],
    out_specs=[
        pl.BlockSpec((R, D), lambda i: (i, 0)),
        pl.BlockSpec((R, D), lambda i: (i, 0)),
    ],
    out_shape=[
        jax.ShapeDtypeStruct((NP, D), jnp.float32),
        jax.ShapeDtypeStruct((NP, D), jnp.float32),
    ],
)


def _tc2_body(accp_ref, aux1_ref, degp_ref, W2_ref, b2_ref, hp2_ref, aux2_ref):
    deg = degp_ref[0, :] + degp_ref[1, :] + 1.0
    dis = lax.rsqrt(deg)[:, None]
    acc = accp_ref[0] + accp_ref[1]
    out1 = jnp.maximum(dis * acc + aux1_ref[...], 0.0)
    h2 = jnp.dot(out1.astype(jnp.bfloat16), W2_ref[...].astype(jnp.bfloat16),
                 preferred_element_type=jnp.float32)
    hp2_ref[...] = h2 * dis
    aux2_ref[...] = h2 * (dis * dis) + b2_ref[...]


_tc2 = pl.pallas_call(
    _tc2_body,
    grid=(G,),
    in_specs=[
        pl.BlockSpec((NCORES, R, D), lambda i: (0, i, 0)),
        pl.BlockSpec((R, D), lambda i: (i, 0)),
        pl.BlockSpec((NCORES, R), lambda i: (0, i)),
        pl.BlockSpec((D, D), lambda i: (0, 0)),
        pl.BlockSpec((1, D), lambda i: (0, 0)),
    ],
    out_specs=[
        pl.BlockSpec((R, D), lambda i: (i, 0)),
        pl.BlockSpec((R, D), lambda i: (i, 0)),
    ],
    out_shape=[
        jax.ShapeDtypeStruct((NP, D), jnp.float32),
        jax.ShapeDtypeStruct((NP, D), jnp.float32),
    ],
)


def _tc3_body(accp_ref, aux2_ref, degp_ref, Wl1_ref, bl1_ref, Wl2_ref,
              bl2_ref, y_ref):
    deg = degp_ref[0, :] + degp_ref[1, :] + 1.0
    dis = lax.rsqrt(deg)[:, None]
    out2 = jnp.maximum(dis * (accp_ref[0] + accp_ref[1]) + aux2_ref[...], 0.0)
    m = jnp.maximum(
        jnp.dot(out2.astype(jnp.bfloat16), Wl1_ref[...].astype(jnp.bfloat16),
                preferred_element_type=jnp.float32)
        + bl1_ref[...], 0.0)
    y_ref[...] = (jnp.dot(m.astype(jnp.bfloat16),
                          Wl2_ref[...].astype(jnp.bfloat16),
                          preferred_element_type=jnp.float32)
                  + bl2_ref[...])


_tc3 = pl.pallas_call(
    _tc3_body,
    grid=(G,),
    in_specs=[
        pl.BlockSpec((NCORES, R, D), lambda i: (0, i, 0)),
        pl.BlockSpec((R, D), lambda i: (i, 0)),
        pl.BlockSpec((NCORES, R), lambda i: (0, i)),
        pl.BlockSpec((D, 10), lambda i: (0, 0)),
        pl.BlockSpec((1, 10), lambda i: (0, 0)),
        pl.BlockSpec((10, 2), lambda i: (0, 0)),
        pl.BlockSpec((1, 2), lambda i: (0, 0)),
    ],
    out_specs=pl.BlockSpec((R, 2), lambda i: (i, 0)),
    out_shape=jax.ShapeDtypeStruct((N, 2), jnp.float32),
)


def kernel(x, edge_index, W1, b1, W2, b2, Wl1, bl1, Wl2, bl2):
    src2d = edge_index[0].astype(jnp.int32).reshape(NROWS_E, CC)
    dst2d = edge_index[1].astype(jnp.int32).reshape(NROWS_E, CC)
    zeros_d = jnp.zeros((RQ, D), jnp.float32)
    zeros_1 = jnp.zeros((ROWS_T,), jnp.float32)

    degp = _sc_degree(dst2d, zeros_1).reshape(NCORES, NP)
    h1 = _tc0(x, W1)
    hp1, aux1 = _tc1(h1, degp, b1.reshape(1, D))
    acc1 = _sc_segsum(hp1, src2d, dst2d, zeros_d)
    W2p = jnp.pad(W2, ((0, 0), (0, D - 20)))
    b2p = jnp.pad(b2, (0, D - 20)).reshape(1, D)
    hp2, aux2 = _tc2(acc1, aux1, degp, W2p, b2p)
    acc2 = _sc_segsum(hp2, src2d, dst2d, zeros_d)
    Wl1p = jnp.pad(Wl1, ((0, D - 20), (0, 0)))
    return _tc3(acc2, aux2, degp, Wl1p, bl1.reshape(1, 10), Wl2,
                bl2.reshape(1, 2))


# speedup vs baseline: 63.9847x; 1.0543x over previous
import functools

import jax
import jax.numpy as jnp
from jax import lax
from jax.experimental import pallas as pl
from jax.experimental.pallas import tpu as pltpu
from jax.experimental.pallas import tpu_sc as plsc

N = 50000
NP = 50176
E = 1600000
D = 32

NCORES = 2
NTILES = 16
NW = NCORES * NTILES
ROWS_T = NP // NTILES
RQ = ROWS_T // 32

CC = 128
NB = 3
NROWS_E = E // CC
ROWS_W = 390
NG = ROWS_W // NB
NTAIL = NROWS_E - NW * ROWS_W

_mesh = plsc.VectorSubcoreMesh(core_axis_name="c", subcore_axis_name="s")


def _gather(tbl_hbm, idxrow, rows, sem):
    return pltpu.make_async_copy(tbl_hbm.at[idxrow], rows, sem)


@functools.partial(
    pl.kernel,
    out_type=jax.ShapeDtypeStruct((NCORES * NP,), jnp.float32),
    mesh=_mesh,
    compiler_params=pltpu.CompilerParams(use_tc_tiling_on_sc=False),
    scratch_types=[
        pltpu.VMEM((NB, 2, CC), jnp.int32),
        pltpu.VMEM((NB, 2, CC), jnp.int32),
        pltpu.VMEM((CC,), jnp.float32),
        pltpu.VMEM((ROWS_T,), jnp.float32),
        pltpu.VMEM_SHARED((NP,), jnp.float32),
        pltpu.SemaphoreType.DMA,
        pltpu.SemaphoreType.DMA,
    ],
)
def _sc_degree(eidx_hbm, zeros_hbm, out_hbm, idx0, idx1, ones_v, stg, deg_sh,
               semi0, semi1):
    c = lax.axis_index("c")
    s = lax.axis_index("s")
    w = c * NTILES + s
    pltpu.sync_copy(zeros_hbm, stg)
    pltpu.sync_copy(stg, deg_sh.at[pl.ds(s * ROWS_T, ROWS_T)])
    for i in range(CC // 16):
        ones_v[pl.ds(i * 16, 16)] = jnp.full((16,), 1.0, jnp.float32)
    plsc.subcore_barrier()

    r0 = w * ROWS_W + jnp.minimum(w, NTAIL)
    pltpu.sync_copy(eidx_hbm.at[pl.ds(r0, NB)], idx0)
    pltpu.async_copy(eidx_hbm.at[pl.ds(r0 + NB, NB)], idx1, semi1)

    def phase(g, idx, oidx, semo, semself):
        @pl.when(g + 1 < NG)
        def _():
            pltpu.make_async_copy(
                eidx_hbm.at[pl.ds(r0 + (g + 1) * NB, NB)], oidx, semo).wait()

        for b in range(NB):
            pltpu.sync_copy(ones_v, deg_sh.at[idx.at[b, 1]], add=True)

        @pl.when(g + 2 < NG)
        def _():
            pltpu.async_copy(
                eidx_hbm.at[pl.ds(r0 + (g + 2) * NB, NB)], idx, semself)

    def pair(k, carry):
        g = 2 * k
        phase(g, idx0, idx1, semi1, semi0)
        phase(g + 1, idx1, idx0, semi0, semi1)
        return carry

    lax.fori_loop(0, NG // 2, pair, 0)

    @pl.when(w < NTAIL)
    def _():
        pltpu.sync_copy(eidx_hbm.at[pl.ds(r0 + ROWS_W, 1)],
                        idx0.at[pl.ds(0, 1)])
        pltpu.sync_copy(ones_v, deg_sh.at[idx0.at[0, 1]], add=True)

    plsc.subcore_barrier()
    pltpu.sync_copy(deg_sh.at[pl.ds(s * ROWS_T, ROWS_T)], stg)
    pltpu.sync_copy(stg, out_hbm.at[pl.ds(c * NP + s * ROWS_T, ROWS_T)])


@functools.partial(
    pl.kernel,
    out_type=jax.ShapeDtypeStruct((NCORES, NP, D), jnp.float32),
    mesh=_mesh,
    compiler_params=pltpu.CompilerParams(use_tc_tiling_on_sc=False),
    scratch_types=[
        pltpu.VMEM((NB, 2, CC), jnp.int32),
        pltpu.VMEM((NB, 2, CC), jnp.int32),
        pltpu.VMEM((NB, CC, D), jnp.float32),
        pltpu.VMEM((NB, CC, D), jnp.float32),
        pltpu.VMEM((RQ, D), jnp.float32),
        pltpu.VMEM_SHARED((NP, D), jnp.float32),
        pltpu.SemaphoreType.DMA,
        pltpu.SemaphoreType.DMA,
        pltpu.SemaphoreType.DMA,
        pltpu.SemaphoreType.DMA,
    ],
)
def _sc_segsum(tbl_hbm, eidx_hbm, zeros_hbm, out_hbm,
               idx0, idx1, rows0, rows1, stg, acc_sh,
               semi0, semi1, semg0, semg1):
    c = lax.axis_index("c")
    s = lax.axis_index("s")
    w = c * NTILES + s
    pltpu.sync_copy(zeros_hbm, stg)
    for q in range(ROWS_T // RQ):
        pltpu.sync_copy(stg, acc_sh.at[pl.ds(s * ROWS_T + q * RQ, RQ)])
    plsc.subcore_barrier()

    r0 = w * ROWS_W + jnp.minimum(w, NTAIL)

    pltpu.sync_copy(eidx_hbm.at[pl.ds(r0, NB)], idx0)
    for b in range(NB):
        _gather(tbl_hbm, idx0.at[b, 0], rows0.at[b], semg0).start()
    pltpu.async_copy(eidx_hbm.at[pl.ds(r0 + NB, NB)], idx1, semi1)

    def phase(g, idx, rows, oidx, orows, semio, semgo, semiself, semgself):
        @pl.when(g + 1 < NG)
        def _():
            pltpu.make_async_copy(
                eidx_hbm.at[pl.ds(r0 + (g + 1) * NB, NB)], oidx, semio).wait()
            for b in range(NB):
                _gather(tbl_hbm, oidx.at[b, 0], orows.at[b], semgo).start()

        for b in range(NB):
            _gather(tbl_hbm, idx.at[b, 0], rows.at[b], semgself).wait()
        for b in range(NB):
            pltpu.sync_copy(rows.at[b], acc_sh.at[idx.at[b, 1]], add=True)

        @pl.when(g + 2 < NG)
        def _():
            pltpu.async_copy(
                eidx_hbm.at[pl.ds(r0 + (g + 2) * NB, NB)], idx, semiself)

    def pair(k, carry):
        g = 2 * k
        phase(g, idx0, rows0, idx1, rows1, semi1, semg1, semi0, semg0)
        phase(g + 1, idx1, rows1, idx0, rows0, semi0, semg0, semi1, semg1)
        return carry

    lax.fori_loop(0, NG // 2, pair, 0)

    @pl.when(w < NTAIL)
    def _():
        pltpu.sync_copy(eidx_hbm.at[pl.ds(r0 + ROWS_W, 1)],
                        idx0.at[pl.ds(0, 1)])
        _gather(tbl_hbm, idx0.at[0, 0], rows0.at[0], semg0).start()
        _gather(tbl_hbm, idx0.at[0, 0], rows0.at[0], semg0).wait()
        pltpu.sync_copy(rows0.at[0], acc_sh.at[idx0.at[0, 1]], add=True)

    plsc.subcore_barrier()
    for q in range(ROWS_T // RQ):
        r = s * ROWS_T + q * RQ
        pltpu.sync_copy(acc_sh.at[pl.ds(r, RQ)], stg)
        pltpu.sync_copy(stg, out_hbm.at[c, pl.ds(r, RQ)])


R = 3584
G = NP // R


def _tc0_body(x_ref, W1_ref, h_ref):
    h_ref[...] = jnp.dot(x_ref[...].astype(jnp.bfloat16),
                         W1_ref[...].astype(jnp.bfloat16),
                         preferred_element_type=jnp.float32)


_tc0 = pl.pallas_call(
    _tc0_body,
    grid=(G,),
    in_specs=[
        pl.BlockSpec((R, 22), lambda i: (i, 0)),
        pl.BlockSpec((22, D), lambda i: (0, 0)),
    ],
    out_specs=pl.BlockSpec((R, D), lambda i: (i, 0)),
    out_shape=jax.ShapeDtypeStruct((NP, D), jnp.float32),
)


def _tc1_body(h_ref, degp_ref, b1_ref, hp_ref, aux_ref):
    deg = degp_ref[0, :] + degp_ref[1, :] + 1.0
    dis = lax.rsqrt(deg)[:, None]
    h = h_ref[...]
    hp_ref[...] = h * dis
    aux_ref[...] = h * (dis * dis) + b1_ref[...]


_tc1 = pl.pallas_call(
    _tc1_body,
    grid=(G,),
    in_specs=[
        pl.BlockSpec((R, D), lambda i: (i, 0)),
        pl.BlockSpec((NCORES, R), lambda i: (0, i)),
        pl.BlockSpec((1, D), lambda i: (0, 0)),
    ],
    out_specs=[
        pl.BlockSpec((R, D), lambda i: (i, 0)),
        pl.BlockSpec((R, D), lambda i: (i, 0)),
    ],
    out_shape=[
        jax.ShapeDtypeStruct((NP, D), jnp.float32),
        jax.ShapeDtypeStruct((NP, D), jnp.float32),
    ],
)


def _tc2_body(accp_ref, aux1_ref, degp_ref, W2_ref, b2_ref, hp2_ref, aux2_ref):
    deg = degp_ref[0, :] + degp_ref[1, :] + 1.0
    dis = lax.rsqrt(deg)[:, None]
    acc = accp_ref[0] + accp_ref[1]
    out1 = jnp.maximum(dis * acc + aux1_ref[...], 0.0)
    h2 = jnp.dot(out1.astype(jnp.bfloat16), W2_ref[...].astype(jnp.bfloat16),
                 preferred_element_type=jnp.float32)
    hp2_ref[...] = h2 * dis
    aux2_ref[...] = h2 * (dis * dis) + b2_ref[...]


_tc2 = pl.pallas_call(
    _tc2_body,
    grid=(G,),
    in_specs=[
        pl.BlockSpec((NCORES, R, D), lambda i: (0, i, 0)),
        pl.BlockSpec((R, D), lambda i: (i, 0)),
        pl.BlockSpec((NCORES, R), lambda i: (0, i)),
        pl.BlockSpec((D, D), lambda i: (0, 0)),
        pl.BlockSpec((1, D), lambda i: (0, 0)),
    ],
    out_specs=[
        pl.BlockSpec((R, D), lambda i: (i, 0)),
        pl.BlockSpec((R, D), lambda i: (i, 0)),
    ],
    out_shape=[
        jax.ShapeDtypeStruct((NP, D), jnp.float32),
        jax.ShapeDtypeStruct((NP, D), jnp.float32),
    ],
)


def _tc3_body(accp_ref, aux2_ref, degp_ref, Wl1_ref, bl1_ref, Wl2_ref,
              bl2_ref, y_ref):
    deg = degp_ref[0, :] + degp_ref[1, :] + 1.0
    dis = lax.rsqrt(deg)[:, None]
    out2 = jnp.maximum(dis * (accp_ref[0] + accp_ref[1]) + aux2_ref[...], 0.0)
    m = jnp.maximum(
        jnp.dot(out2.astype(jnp.bfloat16), Wl1_ref[...].astype(jnp.bfloat16),
                preferred_element_type=jnp.float32)
        + bl1_ref[...], 0.0)
    y_ref[...] = (jnp.dot(m.astype(jnp.bfloat16),
                          Wl2_ref[...].astype(jnp.bfloat16),
                          preferred_element_type=jnp.float32)
                  + bl2_ref[...])


_tc3 = pl.pallas_call(
    _tc3_body,
    grid=(G,),
    in_specs=[
        pl.BlockSpec((NCORES, R, D), lambda i: (0, i, 0)),
        pl.BlockSpec((R, D), lambda i: (i, 0)),
        pl.BlockSpec((NCORES, R), lambda i: (0, i)),
        pl.BlockSpec((D, 10), lambda i: (0, 0)),
        pl.BlockSpec((1, 10), lambda i: (0, 0)),
        pl.BlockSpec((10, 2), lambda i: (0, 0)),
        pl.BlockSpec((1, 2), lambda i: (0, 0)),
    ],
    out_specs=pl.BlockSpec((R, 2), lambda i: (i, 0)),
    out_shape=jax.ShapeDtypeStruct((N, 2), jnp.float32),
)


def kernel(x, edge_index, W1, b1, W2, b2, Wl1, bl1, Wl2, bl2):
    ei = edge_index.astype(jnp.int32)
    eidx = jnp.stack(
        [ei[0].reshape(NROWS_E, CC), ei[1].reshape(NROWS_E, CC)], axis=1)
    zeros_d = jnp.zeros((RQ, D), jnp.float32)
    zeros_1 = jnp.zeros((ROWS_T,), jnp.float32)

    degp = _sc_degree(eidx, zeros_1).reshape(NCORES, NP)
    h1 = _tc0(x, W1)
    hp1, aux1 = _tc1(h1, degp, b1.reshape(1, D))
    acc1 = _sc_segsum(hp1, eidx, zeros_d)
    W2p = jnp.pad(W2, ((0, 0), (0, D - 20)))
    b2p = jnp.pad(b2, (0, D - 20)).reshape(1, D)
    hp2, aux2 = _tc2(acc1, aux1, degp, W2p, b2p)
    acc2 = _sc_segsum(hp2, eidx, zeros_d)
    Wl1p = jnp.pad(Wl1, ((0, D - 20), (0, 0)))
    return _tc3(acc2, aux2, degp, Wl1p, bl1.reshape(1, 10), Wl2,
                bl2.reshape(1, 2))


# speedup vs baseline: 68.3350x; 1.0680x over previous
import functools

import jax
import jax.numpy as jnp
from jax import lax
from jax.experimental import pallas as pl
from jax.experimental.pallas import tpu as pltpu
from jax.experimental.pallas import tpu_sc as plsc

N = 50000
NP = 50176
E = 1600000
D = 32

NCORES = 2
NTILES = 16
NW = NCORES * NTILES
ROWS_T = NP // NTILES
RQ = ROWS_T // 64

CC = 128
NB = 3
NROWS_E = E // CC
ROWS_W = 390
NG = ROWS_W // NB
NTAIL = NROWS_E - NW * ROWS_W

_mesh = plsc.VectorSubcoreMesh(core_axis_name="c", subcore_axis_name="s")


def _gather(tbl_hbm, idxrow, rows, sem):
    return pltpu.make_async_copy(tbl_hbm.at[idxrow], rows, sem)


@functools.partial(
    pl.kernel,
    out_type=jax.ShapeDtypeStruct((NCORES * NP,), jnp.float32),
    mesh=_mesh,
    compiler_params=pltpu.CompilerParams(use_tc_tiling_on_sc=False),
    scratch_types=[
        pltpu.VMEM((NB, 2, CC), jnp.int32),
        pltpu.VMEM((NB, 2, CC), jnp.int32),
        pltpu.VMEM((CC,), jnp.float32),
        pltpu.VMEM((ROWS_T,), jnp.float32),
        pltpu.VMEM_SHARED((NP,), jnp.float32),
        pltpu.SemaphoreType.DMA,
        pltpu.SemaphoreType.DMA,
    ],
)
def _sc_degree(eidx_hbm, zeros_hbm, out_hbm, idx0, idx1, ones_v, stg, deg_sh,
               semi0, semi1):
    c = lax.axis_index("c")
    s = lax.axis_index("s")
    w = c * NTILES + s
    pltpu.sync_copy(zeros_hbm, stg)
    pltpu.sync_copy(stg, deg_sh.at[pl.ds(s * ROWS_T, ROWS_T)])
    for i in range(CC // 16):
        ones_v[pl.ds(i * 16, 16)] = jnp.full((16,), 1.0, jnp.float32)
    plsc.subcore_barrier()

    r0 = w * ROWS_W + jnp.minimum(w, NTAIL)
    pltpu.sync_copy(eidx_hbm.at[pl.ds(r0, NB)], idx0)
    pltpu.async_copy(eidx_hbm.at[pl.ds(r0 + NB, NB)], idx1, semi1)

    def phase(g, idx, oidx, semo, semself):
        @pl.when(g + 1 < NG)
        def _():
            pltpu.make_async_copy(
                eidx_hbm.at[pl.ds(r0 + (g + 1) * NB, NB)], oidx, semo).wait()

        for b in range(NB):
            pltpu.sync_copy(ones_v, deg_sh.at[idx.at[b, 1]], add=True)

        @pl.when(g + 2 < NG)
        def _():
            pltpu.async_copy(
                eidx_hbm.at[pl.ds(r0 + (g + 2) * NB, NB)], idx, semself)

    def pair(k, carry):
        g = 2 * k
        phase(g, idx0, idx1, semi1, semi0)
        phase(g + 1, idx1, idx0, semi0, semi1)
        return carry

    lax.fori_loop(0, NG // 2, pair, 0)

    @pl.when(w < NTAIL)
    def _():
        pltpu.sync_copy(eidx_hbm.at[pl.ds(r0 + ROWS_W, 1)],
                        idx0.at[pl.ds(0, 1)])
        pltpu.sync_copy(ones_v, deg_sh.at[idx0.at[0, 1]], add=True)

    plsc.subcore_barrier()
    pltpu.sync_copy(deg_sh.at[pl.ds(s * ROWS_T, ROWS_T)], stg)
    pltpu.sync_copy(stg, out_hbm.at[pl.ds(c * NP + s * ROWS_T, ROWS_T)])


@functools.partial(
    pl.kernel,
    out_type=jax.ShapeDtypeStruct((NCORES, NP, D), jnp.float32),
    mesh=_mesh,
    compiler_params=pltpu.CompilerParams(use_tc_tiling_on_sc=False),
    scratch_types=[
        pltpu.VMEM((NB, 2, CC), jnp.int32),
        pltpu.VMEM((NB, 2, CC), jnp.int32),
        pltpu.VMEM((NB, 2, CC), jnp.int32),
        pltpu.VMEM((NB, 2, CC), jnp.int32),
        pltpu.VMEM((NB, CC, D), jnp.float32),
        pltpu.VMEM((NB, CC, D), jnp.float32),
        pltpu.VMEM((RQ, D), jnp.float32),
        pltpu.VMEM_SHARED((NP, D), jnp.float32),
        pltpu.SemaphoreType.DMA,
        pltpu.SemaphoreType.DMA,
        pltpu.SemaphoreType.DMA,
        pltpu.SemaphoreType.DMA,
        pltpu.SemaphoreType.DMA,
        pltpu.SemaphoreType.DMA,
    ],
)
def _sc_segsum(tbl_hbm, eidx_hbm, zeros_hbm, out_hbm,
               idx0, idx1, scidx0, scidx1, rows0, rows1, stg, acc_sh,
               semi0, semi1, semg0, semg1, sems0, sems1):
    c = lax.axis_index("c")
    s = lax.axis_index("s")
    w = c * NTILES + s
    pltpu.sync_copy(zeros_hbm, stg)

    def zfill(q, carry):
        pltpu.sync_copy(stg, acc_sh.at[pl.ds(s * ROWS_T + q * RQ, RQ)])
        return carry

    lax.fori_loop(0, ROWS_T // RQ, zfill, 0)
    plsc.subcore_barrier()

    r0 = w * ROWS_W + jnp.minimum(w, NTAIL)

    pltpu.sync_copy(eidx_hbm.at[pl.ds(r0, NB)], idx0)
    for b in range(NB):
        _gather(tbl_hbm, idx0.at[b, 0], rows0.at[b], semg0).start()
    pltpu.async_copy(eidx_hbm.at[pl.ds(r0 + NB, NB)], idx1, semi1)

    def phase(g, idx, scidx, rows, oidx, oscidx, orows,
              semio, semgo, semso, semiself, semgself, semsself):
        @pl.when(jnp.logical_and(g >= 1, g + 1 < NG))
        def _():
            for b in range(NB):
                pltpu.make_async_copy(
                    orows.at[b], acc_sh.at[oscidx.at[b, 1]], semso).wait()

        @pl.when(g + 1 < NG)
        def _():
            pltpu.make_async_copy(
                eidx_hbm.at[pl.ds(r0 + (g + 1) * NB, NB)], oidx, semio).wait()
            for b in range(NB):
                _gather(tbl_hbm, oidx.at[b, 0], orows.at[b], semgo).start()

        for b in range(NB):
            _gather(tbl_hbm, idx.at[b, 0], rows.at[b], semgself).wait()
        for b in range(NB):
            for i in range(CC // 16):
                scidx[b, 1, pl.ds(i * 16, 16)] = idx[b, 1, pl.ds(i * 16, 16)]
        for b in range(NB):
            pltpu.async_copy(rows.at[b], acc_sh.at[scidx.at[b, 1]],
                             semsself, add=True)

        @pl.when(g + 2 < NG)
        def _():
            pltpu.async_copy(
                eidx_hbm.at[pl.ds(r0 + (g + 2) * NB, NB)], idx, semiself)

    def pair(k, carry):
        g = 2 * k
        phase(g, idx0, scidx0, rows0, idx1, scidx1, rows1,
              semi1, semg1, sems1, semi0, semg0, sems0)
        phase(g + 1, idx1, scidx1, rows1, idx0, scidx0, rows0,
              semi0, semg0, sems0, semi1, semg1, sems1)
        return carry

    lax.fori_loop(0, NG // 2, pair, 0)
    for b in range(NB):
        pltpu.make_async_copy(
            rows0.at[b], acc_sh.at[scidx0.at[b, 1]], sems0).wait()
    for b in range(NB):
        pltpu.make_async_copy(
            rows1.at[b], acc_sh.at[scidx1.at[b, 1]], sems1).wait()

    @pl.when(w < NTAIL)
    def _():
        pltpu.sync_copy(eidx_hbm.at[pl.ds(r0 + ROWS_W, 1)],
                        idx0.at[pl.ds(0, 1)])
        _gather(tbl_hbm, idx0.at[0, 0], rows0.at[0], semg0).start()
        _gather(tbl_hbm, idx0.at[0, 0], rows0.at[0], semg0).wait()
        pltpu.sync_copy(rows0.at[0], acc_sh.at[idx0.at[0, 1]], add=True)

    plsc.subcore_barrier()

    def rdback(q, carry):
        r = s * ROWS_T + q * RQ
        pltpu.sync_copy(acc_sh.at[pl.ds(r, RQ)], stg)
        pltpu.sync_copy(stg, out_hbm.at[c, pl.ds(r, RQ)])
        return carry

    lax.fori_loop(0, ROWS_T // RQ, rdback, 0)


R = 3584
G = NP // R


def _tc0_body(x_ref, W1_ref, h_ref):
    h_ref[...] = jnp.dot(x_ref[...].astype(jnp.bfloat16),
                         W1_ref[...].astype(jnp.bfloat16),
                         preferred_element_type=jnp.float32)


_tc0 = pl.pallas_call(
    _tc0_body,
    grid=(G,),
    in_specs=[
        pl.BlockSpec((R, 22), lambda i: (i, 0)),
        pl.BlockSpec((22, D), lambda i: (0, 0)),
    ],
    out_specs=pl.BlockSpec((R, D), lambda i: (i, 0)),
    out_shape=jax.ShapeDtypeStruct((NP, D), jnp.float32),
)


def _tc1_body(h_ref, degp_ref, b1_ref, hp_ref, aux_ref):
    deg = degp_ref[0, :] + degp_ref[1, :] + 1.0
    dis = lax.rsqrt(deg)[:, None]
    h = h_ref[...]
    hp_ref[...] = h * dis
    aux_ref[...] = h * (dis * dis) + b1_ref[...]


_tc1 = pl.pallas_call(
    _tc1_body,
    grid=(G,),
    in_specs=[
        pl.BlockSpec((R, D), lambda i: (i, 0)),
        pl.BlockSpec((NCORES, R), lambda i: (0, i)),
        pl.BlockSpec((1, D), lambda i: (0, 0)),
    ],
    out_specs=[
        pl.BlockSpec((R, D), lambda i: (i, 0)),
        pl.BlockSpec((R, D), lambda i: (i, 0)),
    ],
    out_shape=[
        jax.ShapeDtypeStruct((NP, D), jnp.float32),
        jax.ShapeDtypeStruct((NP, D), jnp.float32),
    ],
)


def _tc2_body(accp_ref, aux1_ref, degp_ref, W2_ref, b2_ref, hp2_ref, aux2_ref):
    deg = degp_ref[0, :] + degp_ref[1, :] + 1.0
    dis = lax.rsqrt(deg)[:, None]
    acc = accp_ref[0] + accp_ref[1]
    out1 = jnp.maximum(dis * acc + aux1_ref[...], 0.0)
    h2 = jnp.dot(out1.astype(jnp.bfloat16), W2_ref[...].astype(jnp.bfloat16),
                 preferred_element_type=jnp.float32)
    hp2_ref[...] = h2 * dis
    aux2_ref[...] = h2 * (dis * dis) + b2_ref[...]


_tc2 = pl.pallas_call(
    _tc2_body,
    grid=(G,),
    in_specs=[
        pl.BlockSpec((NCORES, R, D), lambda i: (0, i, 0)),
        pl.BlockSpec((R, D), lambda i: (i, 0)),
        pl.BlockSpec((NCORES, R), lambda i: (0, i)),
        pl.BlockSpec((D, D), lambda i: (0, 0)),
        pl.BlockSpec((1, D), lambda i: (0, 0)),
    ],
    out_specs=[
        pl.BlockSpec((R, D), lambda i: (i, 0)),
        pl.BlockSpec((R, D), lambda i: (i, 0)),
    ],
    out_shape=[
        jax.ShapeDtypeStruct((NP, D), jnp.float32),
        jax.ShapeDtypeStruct((NP, D), jnp.float32),
    ],
)


def _tc3_body(accp_ref, aux2_ref, degp_ref, Wl1_ref, bl1_ref, Wl2_ref,
              bl2_ref, y_ref):
    deg = degp_ref[0, :] + degp_ref[1, :] + 1.0
    dis = lax.rsqrt(deg)[:, None]
    out2 = jnp.maximum(dis * (accp_ref[0] + accp_ref[1]) + aux2_ref[...], 0.0)
    m = jnp.maximum(
        jnp.dot(out2.astype(jnp.bfloat16), Wl1_ref[...].astype(jnp.bfloat16),
                preferred_element_type=jnp.float32)
        + bl1_ref[...], 0.0)
    y_ref[...] = (jnp.dot(m.astype(jnp.bfloat16),
                          Wl2_ref[...].astype(jnp.bfloat16),
                          preferred_element_type=jnp.float32)
                  + bl2_ref[...])


_tc3 = pl.pallas_call(
    _tc3_body,
    grid=(G,),
    in_specs=[
        pl.BlockSpec((NCORES, R, D), lambda i: (0, i, 0)),
        pl.BlockSpec((R, D), lambda i: (i, 0)),
        pl.BlockSpec((NCORES, R), lambda i: (0, i)),
        pl.BlockSpec((D, 10), lambda i: (0, 0)),
        pl.BlockSpec((1, 10), lambda i: (0, 0)),
        pl.BlockSpec((10, 2), lambda i: (0, 0)),
        pl.BlockSpec((1, 2), lambda i: (0, 0)),
    ],
    out_specs=pl.BlockSpec((R, 2), lambda i: (i, 0)),
    out_shape=jax.ShapeDtypeStruct((N, 2), jnp.float32),
)


def kernel(x, edge_index, W1, b1, W2, b2, Wl1, bl1, Wl2, bl2):
    ei = edge_index.astype(jnp.int32)
    eidx = jnp.stack(
        [ei[0].reshape(NROWS_E, CC), ei[1].reshape(NROWS_E, CC)], axis=1)
    zeros_d = jnp.zeros((RQ, D), jnp.float32)
    zeros_1 = jnp.zeros((ROWS_T,), jnp.float32)

    degp = _sc_degree(eidx, zeros_1).reshape(NCORES, NP)
    h1 = _tc0(x, W1)
    hp1, aux1 = _tc1(h1, degp, b1.reshape(1, D))
    acc1 = _sc_segsum(hp1, eidx, zeros_d)
    W2p = jnp.pad(W2, ((0, 0), (0, D - 20)))
    b2p = jnp.pad(b2, (0, D - 20)).reshape(1, D)
    hp2, aux2 = _tc2(acc1, aux1, degp, W2p, b2p)
    acc2 = _sc_segsum(hp2, eidx, zeros_d)
    Wl1p = jnp.pad(Wl1, ((0, D - 20), (0, 0)))
    return _tc3(acc2, aux2, degp, Wl1p, bl1.reshape(1, 10), Wl2,
                bl2.reshape(1, 2))


# speedup vs baseline: 71.7991x; 1.0507x over previous
import functools

import jax
import jax.numpy as jnp
from jax import lax
from jax.experimental import pallas as pl
from jax.experimental.pallas import tpu as pltpu
from jax.experimental.pallas import tpu_sc as plsc

N = 50000
NP = 50176
E = 1600000
D = 32

NCORES = 2
NTILES = 16
NW = NCORES * NTILES
ROWS_T = NP // NTILES
RQ = ROWS_T // 64

CC = 128
NB = 3
NROWS_E = E // CC
ROWS_W = 390
NG = ROWS_W // NB
NTAIL = NROWS_E - NW * ROWS_W

_mesh = plsc.VectorSubcoreMesh(core_axis_name="c", subcore_axis_name="s")


def _gather(tbl_hbm, idxrow, rows, sem):
    return pltpu.make_async_copy(tbl_hbm.at[idxrow], rows, sem)


@functools.partial(
    pl.kernel,
    out_type=jax.ShapeDtypeStruct((NCORES * NP,), jnp.float32),
    mesh=_mesh,
    compiler_params=pltpu.CompilerParams(use_tc_tiling_on_sc=False),
    scratch_types=[
        pltpu.VMEM((NB, 2, CC), jnp.int32),
        pltpu.VMEM((NB, 2, CC), jnp.int32),
        pltpu.VMEM((NB, 2, CC), jnp.int32),
        pltpu.VMEM((NB, 2, CC), jnp.int32),
        pltpu.VMEM((CC,), jnp.float32),
        pltpu.VMEM((ROWS_T,), jnp.float32),
        pltpu.VMEM_SHARED((NP,), jnp.float32),
        pltpu.SemaphoreType.DMA,
        pltpu.SemaphoreType.DMA,
        pltpu.SemaphoreType.DMA,
        pltpu.SemaphoreType.DMA,
    ],
)
def _sc_degree(eidx_hbm, zeros_hbm, out_hbm, idx0, idx1, scidx0, scidx1,
               ones_v, stg, deg_sh, semi0, semi1, sems0, sems1):
    c = lax.axis_index("c")
    s = lax.axis_index("s")
    w = c * NTILES + s
    pltpu.sync_copy(zeros_hbm, stg)
    pltpu.sync_copy(stg, deg_sh.at[pl.ds(s * ROWS_T, ROWS_T)])
    for i in range(CC // 16):
        ones_v[pl.ds(i * 16, 16)] = jnp.full((16,), 1.0, jnp.float32)
    plsc.subcore_barrier()

    r0 = w * ROWS_W + jnp.minimum(w, NTAIL)
    pltpu.sync_copy(eidx_hbm.at[pl.ds(r0, NB)], idx0)
    pltpu.async_copy(eidx_hbm.at[pl.ds(r0 + NB, NB)], idx1, semi1)

    def phase(g, idx, scidx, oidx, oscidx, semo, semso, semself, semsself):
        @pl.when(jnp.logical_and(g >= 1, g + 1 < NG))
        def _():
            for b in range(NB):
                pltpu.make_async_copy(
                    ones_v, deg_sh.at[oscidx.at[b, 1]], semso).wait()

        @pl.when(g + 1 < NG)
        def _():
            pltpu.make_async_copy(
                eidx_hbm.at[pl.ds(r0 + (g + 1) * NB, NB)], oidx, semo).wait()

        for b in range(NB):
            for i in range(CC // 16):
                scidx[b, 1, pl.ds(i * 16, 16)] = idx[b, 1, pl.ds(i * 16, 16)]
        for b in range(NB):
            pltpu.async_copy(ones_v, deg_sh.at[scidx.at[b, 1]], semsself,
                             add=True)

        @pl.when(g + 2 < NG)
        def _():
            pltpu.async_copy(
                eidx_hbm.at[pl.ds(r0 + (g + 2) * NB, NB)], idx, semself)

    def pair(k, carry):
        g = 2 * k
        phase(g, idx0, scidx0, idx1, scidx1, semi1, sems1, semi0, sems0)
        phase(g + 1, idx1, scidx1, idx0, scidx0, semi0, sems0, semi1, sems1)
        return carry

    lax.fori_loop(0, NG // 2, pair, 0)
    for b in range(NB):
        pltpu.make_async_copy(ones_v, deg_sh.at[scidx0.at[b, 1]],
                              sems0).wait()
    for b in range(NB):
        pltpu.make_async_copy(ones_v, deg_sh.at[scidx1.at[b, 1]],
                              sems1).wait()

    @pl.when(w < NTAIL)
    def _():
        pltpu.sync_copy(eidx_hbm.at[pl.ds(r0 + ROWS_W, 1)],
                        idx0.at[pl.ds(0, 1)])
        pltpu.sync_copy(ones_v, deg_sh.at[idx0.at[0, 1]], add=True)

    plsc.subcore_barrier()
    pltpu.sync_copy(deg_sh.at[pl.ds(s * ROWS_T, ROWS_T)], stg)
    pltpu.sync_copy(stg, out_hbm.at[pl.ds(c * NP + s * ROWS_T, ROWS_T)])


@functools.partial(
    pl.kernel,
    out_type=jax.ShapeDtypeStruct((NCORES, NP, D), jnp.float32),
    mesh=_mesh,
    compiler_params=pltpu.CompilerParams(use_tc_tiling_on_sc=False),
    scratch_types=[
        pltpu.VMEM((NB, 2, CC), jnp.int32),
        pltpu.VMEM((NB, 2, CC), jnp.int32),
        pltpu.VMEM((NB, 2, CC), jnp.int32),
        pltpu.VMEM((NB, 2, CC), jnp.int32),
        pltpu.VMEM((NB, CC, D), jnp.float32),
        pltpu.VMEM((NB, CC, D), jnp.float32),
        pltpu.VMEM((RQ, D), jnp.float32),
        pltpu.VMEM_SHARED((NP, D), jnp.float32),
        pltpu.SemaphoreType.DMA,
        pltpu.SemaphoreType.DMA,
        pltpu.SemaphoreType.DMA,
        pltpu.SemaphoreType.DMA,
        pltpu.SemaphoreType.DMA,
        pltpu.SemaphoreType.DMA,
    ],
)
def _sc_segsum(tbl_hbm, eidx_hbm, zeros_hbm, out_hbm,
               idx0, idx1, scidx0, scidx1, rows0, rows1, stg, acc_sh,
               semi0, semi1, semg0, semg1, sems0, sems1):
    c = lax.axis_index("c")
    s = lax.axis_index("s")
    w = c * NTILES + s
    pltpu.sync_copy(zeros_hbm, stg)

    def zfill(q, carry):
        pltpu.sync_copy(stg, acc_sh.at[pl.ds(s * ROWS_T + q * RQ, RQ)])
        return carry

    lax.fori_loop(0, ROWS_T // RQ, zfill, 0)
    plsc.subcore_barrier()

    r0 = w * ROWS_W + jnp.minimum(w, NTAIL)

    pltpu.sync_copy(eidx_hbm.at[pl.ds(r0, NB)], idx0)
    for b in range(NB):
        _gather(tbl_hbm, idx0.at[b, 0], rows0.at[b], semg0).start()
    pltpu.async_copy(eidx_hbm.at[pl.ds(r0 + NB, NB)], idx1, semi1)

    def phase(g, idx, scidx, rows, oidx, oscidx, orows,
              semio, semgo, semso, semiself, semgself, semsself):
        @pl.when(jnp.logical_and(g >= 1, g + 1 < NG))
        def _():
            for b in range(NB):
                pltpu.make_async_copy(
                    orows.at[b], acc_sh.at[oscidx.at[b, 1]], semso).wait()

        @pl.when(g + 1 < NG)
        def _():
            pltpu.make_async_copy(
                eidx_hbm.at[pl.ds(r0 + (g + 1) * NB, NB)], oidx, semio).wait()
            for b in range(NB):
                _gather(tbl_hbm, oidx.at[b, 0], orows.at[b], semgo).start()

        for b in range(NB):
            _gather(tbl_hbm, idx.at[b, 0], rows.at[b], semgself).wait()
        for b in range(NB):
            for i in range(CC // 16):
                scidx[b, 1, pl.ds(i * 16, 16)] = idx[b, 1, pl.ds(i * 16, 16)]
        for b in range(NB):
            pltpu.async_copy(rows.at[b], acc_sh.at[scidx.at[b, 1]],
                             semsself, add=True)

        @pl.when(g + 2 < NG)
        def _():
            pltpu.async_copy(
                eidx_hbm.at[pl.ds(r0 + (g + 2) * NB, NB)], idx, semiself)

    def pair(k, carry):
        g = 2 * k
        phase(g, idx0, scidx0, rows0, idx1, scidx1, rows1,
              semi1, semg1, sems1, semi0, semg0, sems0)
        phase(g + 1, idx1, scidx1, rows1, idx0, scidx0, rows0,
              semi0, semg0, sems0, semi1, semg1, sems1)
        return carry

    lax.fori_loop(0, NG // 2, pair, 0)
    for b in range(NB):
        pltpu.make_async_copy(
            rows0.at[b], acc_sh.at[scidx0.at[b, 1]], sems0).wait()
    for b in range(NB):
        pltpu.make_async_copy(
            rows1.at[b], acc_sh.at[scidx1.at[b, 1]], sems1).wait()

    @pl.when(w < NTAIL)
    def _():
        pltpu.sync_copy(eidx_hbm.at[pl.ds(r0 + ROWS_W, 1)],
                        idx0.at[pl.ds(0, 1)])
        _gather(tbl_hbm, idx0.at[0, 0], rows0.at[0], semg0).start()
        _gather(tbl_hbm, idx0.at[0, 0], rows0.at[0], semg0).wait()
        pltpu.sync_copy(rows0.at[0], acc_sh.at[idx0.at[0, 1]], add=True)

    plsc.subcore_barrier()

    def rdback(q, carry):
        r = s * ROWS_T + q * RQ
        pltpu.sync_copy(acc_sh.at[pl.ds(r, RQ)], stg)
        pltpu.sync_copy(stg, out_hbm.at[c, pl.ds(r, RQ)])
        return carry

    lax.fori_loop(0, ROWS_T // RQ, rdback, 0)


R = 3584
G = NP // R


def _tc0_body(x_ref, W1_ref, h_ref):
    h_ref[...] = jnp.dot(x_ref[...].astype(jnp.bfloat16),
                         W1_ref[...].astype(jnp.bfloat16),
                         preferred_element_type=jnp.float32)


_tc0 = pl.pallas_call(
    _tc0_body,
    grid=(G,),
    in_specs=[
        pl.BlockSpec((R, 22), lambda i: (i, 0)),
        pl.BlockSpec((22, D), lambda i: (0, 0)),
    ],
    out_specs=pl.BlockSpec((R, D), lambda i: (i, 0)),
    out_shape=jax.ShapeDtypeStruct((NP, D), jnp.float32),
)


def _tc1_body(h_ref, degp_ref, b1_ref, hp_ref, aux_ref):
    deg = degp_ref[0, :] + degp_ref[1, :] + 1.0
    dis = lax.rsqrt(deg)[:, None]
    h = h_ref[...]
    hp_ref[...] = h * dis
    aux_ref[...] = h * (dis * dis) + b1_ref[...]


_tc1 = pl.pallas_call(
    _tc1_body,
    grid=(G,),
    in_specs=[
        pl.BlockSpec((R, D), lambda i: (i, 0)),
        pl.BlockSpec((NCORES, R), lambda i: (0, i)),
        pl.BlockSpec((1, D), lambda i: (0, 0)),
    ],
    out_specs=[
        pl.BlockSpec((R, D), lambda i: (i, 0)),
        pl.BlockSpec((R, D), lambda i: (i, 0)),
    ],
    out_shape=[
        jax.ShapeDtypeStruct((NP, D), jnp.float32),
        jax.ShapeDtypeStruct((NP, D), jnp.float32),
    ],
)


def _tc2_body(accp_ref, aux1_ref, degp_ref, W2_ref, b2_ref, hp2_ref, aux2_ref):
    deg = degp_ref[0, :] + degp_ref[1, :] + 1.0
    dis = lax.rsqrt(deg)[:, None]
    acc = accp_ref[0] + accp_ref[1]
    out1 = jnp.maximum(dis * acc + aux1_ref[...], 0.0)
    h2 = jnp.dot(out1.astype(jnp.bfloat16), W2_ref[...].astype(jnp.bfloat16),
                 preferred_element_type=jnp.float32)
    hp2_ref[...] = h2 * dis
    aux2_ref[...] = h2 * (dis * dis) + b2_ref[...]


_tc2 = pl.pallas_call(
    _tc2_body,
    grid=(G,),
    in_specs=[
        pl.BlockSpec((NCORES, R, D), lambda i: (0, i, 0)),
        pl.BlockSpec((R, D), lambda i: (i, 0)),
        pl.BlockSpec((NCORES, R), lambda i: (0, i)),
        pl.BlockSpec((D, D), lambda i: (0, 0)),
        pl.BlockSpec((1, D), lambda i: (0, 0)),
    ],
    out_specs=[
        pl.BlockSpec((R, D), lambda i: (i, 0)),
        pl.BlockSpec((R, D), lambda i: (i, 0)),
    ],
    out_shape=[
        jax.ShapeDtypeStruct((NP, D), jnp.float32),
        jax.ShapeDtypeStruct((NP, D), jnp.float32),
    ],
)


def _tc3_body(accp_ref, aux2_ref, degp_ref, Wl1_ref, bl1_ref, Wl2_ref,
              bl2_ref, y_ref):
    deg = degp_ref[0, :] + degp_ref[1, :] + 1.0
    dis = lax.rsqrt(deg)[:, None]
    out2 = jnp.maximum(dis * (accp_ref[0] + accp_ref[1]) + aux2_ref[...], 0.0)
    m = jnp.maximum(
        jnp.dot(out2.astype(jnp.bfloat16), Wl1_ref[...].astype(jnp.bfloat16),
                preferred_element_type=jnp.float32)
        + bl1_ref[...], 0.0)
    y_ref[...] = (jnp.dot(m.astype(jnp.bfloat16),
                          Wl2_ref[...].astype(jnp.bfloat16),
                          preferred_element_type=jnp.float32)
                  + bl2_ref[...])


_tc3 = pl.pallas_call(
    _tc3_body,
    grid=(G,),
    in_specs=[
        pl.BlockSpec((NCORES, R, D), lambda i: (0, i, 0)),
        pl.BlockSpec((R, D), lambda i: (i, 0)),
        pl.BlockSpec((NCORES, R), lambda i: (0, i)),
        pl.BlockSpec((D, 10), lambda i: (0, 0)),
        pl.BlockSpec((1, 10), lambda i: (0, 0)),
        pl.BlockSpec((10, 2), lambda i: (0, 0)),
        pl.BlockSpec((1, 2), lambda i: (0, 0)),
    ],
    out_specs=pl.BlockSpec((R, 2), lambda i: (i, 0)),
    out_shape=jax.ShapeDtypeStruct((N, 2), jnp.float32),
)


def kernel(x, edge_index, W1, b1, W2, b2, Wl1, bl1, Wl2, bl2):
    ei = edge_index.astype(jnp.int32)
    eidx = jnp.stack(
        [ei[0].reshape(NROWS_E, CC), ei[1].reshape(NROWS_E, CC)], axis=1)
    zeros_d = jnp.zeros((RQ, D), jnp.float32)
    zeros_1 = jnp.zeros((ROWS_T,), jnp.float32)

    degp = _sc_degree(eidx, zeros_1).reshape(NCORES, NP)
    h1 = _tc0(x, W1)
    hp1, aux1 = _tc1(h1, degp, b1.reshape(1, D))
    acc1 = _sc_segsum(hp1, eidx, zeros_d)
    W2p = jnp.pad(W2, ((0, 0), (0, D - 20)))
    b2p = jnp.pad(b2, (0, D - 20)).reshape(1, D)
    hp2, aux2 = _tc2(acc1, aux1, degp, W2p, b2p)
    acc2 = _sc_segsum(hp2, eidx, zeros_d)
    Wl1p = jnp.pad(Wl1, ((0, D - 20), (0, 0)))
    return _tc3(acc2, aux2, degp, Wl1p, bl1.reshape(1, 10), Wl2,
                bl2.reshape(1, 2))
